# baseline (speedup 1.0000x reference)
import functools

import jax
import jax.numpy as jnp
from jax import lax
from jax.experimental import pallas as pl
from jax.experimental.pallas import tpu as pltpu
from jax.experimental.pallas import tpu_sc as plsc

N = 10000
NP = 10240
D = 128
E = 320000
NC, NS = 2, 16
NW = NC * NS
CB = 128
NCH = 80
TCH = NW * NCH
EP = TCH * CB
ACH = 48
BCH = NCH * 2 - ACH
NCH_MAX = max(ACH, BCH)
RPT = NP // NS
DW = 128

_MESH = plsc.VectorSubcoreMesh(
    core_axis_name="c", subcore_axis_name="s", num_cores=NC, num_subcores=NS
)


def _prop_body(g_hbm, src_hbm, dst_hbm, zeros_hbm, out_hbm,
               src_v, dst_v, rows_v, acc_sh, gsem):
    c = lax.axis_index("c")
    s = lax.axis_index("s")
    nch = jnp.where(c == 0, ACH, BCH)
    off = jnp.where(c == 0, s * ACH, 16 * ACH + s * BCH)
    pltpu.sync_copy(zeros_hbm, acc_sh.at[pl.ds(s * RPT, RPT)])
    pltpu.sync_copy(src_hbm.at[pl.ds(off, NCH_MAX)], src_v)
    pltpu.sync_copy(dst_hbm.at[pl.ds(off, NCH_MAX)], dst_v)
    plsc.subcore_barrier()

    def body(j, carry):
        pltpu.async_copy(g_hbm.at[src_v.at[j]], rows_v, gsem).wait()
        pltpu.sync_copy(rows_v, acc_sh.at[dst_v.at[j]], add=True)
        return carry

    lax.fori_loop(0, nch, body, 0)
    plsc.subcore_barrier()
    pltpu.sync_copy(
        acc_sh.at[pl.ds(s * RPT, RPT)], out_hbm.at[c, pl.ds(s * RPT, RPT)]
    )


_prop = pl.kernel(
    _prop_body,
    out_type=jax.ShapeDtypeStruct((NC, NP, D), jnp.float32),
    mesh=_MESH,
    scratch_types=[
        pltpu.VMEM((NCH_MAX, CB), jnp.int32),
        pltpu.VMEM((NCH_MAX, CB), jnp.int32),
        pltpu.VMEM((CB, D), jnp.float32),
        pltpu.VMEM_SHARED((NP, D), jnp.float32),
        pltpu.SemaphoreType.DMA,
    ],
)


def _deg_body(dst_hbm, ones_hbm, zeros_hbm, out_hbm, dst_v, ones_v, acc_sh):
    c = lax.axis_index("c")
    s = lax.axis_index("s")
    wid = c * NS + s
    pltpu.sync_copy(zeros_hbm, acc_sh.at[pl.ds(s * RPT, RPT)])
    pltpu.sync_copy(ones_hbm, ones_v)
    pltpu.sync_copy(dst_hbm.at[pl.ds(wid * NCH, NCH)], dst_v)
    plsc.subcore_barrier()

    def body(j, carry):
        pltpu.sync_copy(ones_v, acc_sh.at[dst_v.at[j]], add=True)
        return carry

    lax.fori_loop(0, NCH, body, 0)
    plsc.subcore_barrier()
    pltpu.sync_copy(
        acc_sh.at[pl.ds(s * RPT, RPT)], out_hbm.at[c, pl.ds(s * RPT, RPT)]
    )


_deg = pl.kernel(
    _deg_body,
    out_type=jax.ShapeDtypeStruct((NC, NP, D), jnp.float32),
    mesh=_MESH,
    scratch_types=[
        pltpu.VMEM((NCH, CB), jnp.int32),
        pltpu.VMEM((CB, D), jnp.float32),
        pltpu.VMEM_SHARED((NP, D), jnp.float32),
    ],
)


BM = 1024


def _dinv_deginv(dp):
    deg = 1.0 + dp[0, :, 0] + dp[1, :, 0]
    return lax.rsqrt(deg), 1.0 / deg


def _dense0_body(x_ref, w_ref, dp_ref, t_ref, g_ref):
    dinv, _ = _dinv_deginv(dp_ref[...])
    t = jnp.dot(x_ref[...], w_ref[...], preferred_element_type=jnp.float32)
    t_ref[...] = t
    g_ref[...] = t * dinv[:, None]


def _mid_body(acc_ref, t_ref, b_ref, dp_ref, w_ref, tn_ref, gn_ref):
    dinv, deginv = _dinv_deginv(dp_ref[...])
    agg = ((acc_ref[0] + acc_ref[1]) * dinv[:, None]
           + t_ref[...] * deginv[:, None] + b_ref[...])
    h = jax.nn.sigmoid(agg)
    t = jnp.dot(h, w_ref[...], preferred_element_type=jnp.float32)
    tn_ref[...] = t
    gn_ref[...] = t * dinv[:, None]


def _fin_body(acc_ref, t_ref, b_ref, dp_ref, o_ref):
    dinv, deginv = _dinv_deginv(dp_ref[...])
    o_ref[...] = ((acc_ref[0] + acc_ref[1]) * dinv[:, None]
                  + t_ref[...] * deginv[:, None] + b_ref[...])


_spec_rows = pl.BlockSpec((BM, D), lambda i: (i, 0))
_spec_w = pl.BlockSpec((D, D), lambda i: (0, 0))
_spec_b = pl.BlockSpec((1, D), lambda i: (0, 0))
_spec_dp = pl.BlockSpec((NC, BM, DW), lambda i: (0, i, 0))
_spec_acc = pl.BlockSpec((NC, BM, D), lambda i: (0, i, 0))
_GRID = (NP // BM,)
_row_ty = jax.ShapeDtypeStruct((NP, D), jnp.float32)

_dense0 = pl.pallas_call(
    _dense0_body,
    grid=_GRID,
    in_specs=[_spec_rows, _spec_w, _spec_dp],
    out_specs=[_spec_rows, _spec_rows],
    out_shape=[_row_ty, _row_ty],
)

_mid = pl.pallas_call(
    _mid_body,
    grid=_GRID,
    in_specs=[_spec_acc, _spec_rows, _spec_b, _spec_dp, _spec_w],
    out_specs=[_spec_rows, _spec_rows],
    out_shape=[_row_ty, _row_ty],
)

_fin = pl.pallas_call(
    _fin_body,
    grid=_GRID,
    in_specs=[_spec_acc, _spec_rows, _spec_b, _spec_dp],
    out_specs=_spec_rows,
    out_shape=_row_ty,
)


@jax.jit
def kernel(x, edge_attr, edge_index, W0, b0, W1, b1, W2, b2):
    del edge_attr
    src = edge_index[0]
    dst = edge_index[1]
    pad = EP - E
    pad_dst = N + jnp.arange(pad, dtype=jnp.int32) % (NP - N)
    src_p = jnp.concatenate(
        [src, jnp.zeros((pad,), jnp.int32)]).reshape(TCH, CB)
    dst_p = jnp.concatenate([dst, pad_dst]).reshape(TCH, CB)
    x_p = jnp.pad(x, ((0, NP - N), (0, 0)))
    zeros_rows = jnp.zeros((RPT, D), jnp.float32)
    ones_rows = jnp.ones((CB, D), jnp.float32)

    dp = _deg(dst_p, ones_rows, zeros_rows)
    t0, g0 = _dense0(x_p, W0, dp)
    a0 = _prop(g0, src_p, dst_p, zeros_rows)
    t1, g1 = _mid(a0, t0, b0.reshape(1, D), dp, W1)
    a1 = _prop(g1, src_p, dst_p, zeros_rows)
    t2, g2 = _mid(a1, t1, b1.reshape(1, D), dp, W2)
    a2 = _prop(g2, src_p, dst_p, zeros_rows)
    out = _fin(a2, t2, b2.reshape(1, D), dp)
    return out[:N]


# speedup vs baseline: 1.2341x; 1.2341x over previous
import functools

import jax
import jax.numpy as jnp
from jax import lax
from jax.experimental import pallas as pl
from jax.experimental.pallas import tpu as pltpu
from jax.experimental.pallas import tpu_sc as plsc

N = 10000
NP = 10240
D = 128
E = 320000
NC, NS = 2, 16
NW = NC * NS
CB = 128
NCH = 80
TCH = NW * NCH
EP = TCH * CB
ACH = 112
BCH = NCH * 2 - ACH
NCH_MAX = max(ACH, BCH)
RPT = NP // NS
DW = 128

_MESH = plsc.VectorSubcoreMesh(
    core_axis_name="c", subcore_axis_name="s", num_cores=NC, num_subcores=NS
)


def _prop_body(g_hbm, src_hbm, dst_hbm, zeros_hbm, out_hbm,
               src_v, dst_v, rows_v, acc_sh, gsem):
    c = lax.axis_index("c")
    s = lax.axis_index("s")
    nch = jnp.where(c == 0, ACH, BCH)
    off = jnp.where(c == 0, s * ACH, 16 * ACH + s * BCH)
    pltpu.sync_copy(zeros_hbm, acc_sh.at[pl.ds(s * RPT, RPT)])
    pltpu.sync_copy(src_hbm.at[pl.ds(off, NCH_MAX)], src_v)
    pltpu.sync_copy(dst_hbm.at[pl.ds(off, NCH_MAX)], dst_v)
    plsc.subcore_barrier()

    def body(j, carry):
        pltpu.async_copy(g_hbm.at[src_v.at[j]], rows_v, gsem).wait()
        pltpu.sync_copy(rows_v, acc_sh.at[dst_v.at[j]], add=True)
        return carry

    lax.fori_loop(0, nch, body, 0)
    plsc.subcore_barrier()
    pltpu.sync_copy(
        acc_sh.at[pl.ds(s * RPT, RPT)], out_hbm.at[c, pl.ds(s * RPT, RPT)]
    )


_prop = pl.kernel(
    _prop_body,
    out_type=jax.ShapeDtypeStruct((NC, NP, D), jnp.float32),
    mesh=_MESH,
    scratch_types=[
        pltpu.VMEM((NCH_MAX, CB), jnp.int32),
        pltpu.VMEM((NCH_MAX, CB), jnp.int32),
        pltpu.VMEM((CB, D), jnp.float32),
        pltpu.VMEM_SHARED((NP, D), jnp.float32),
        pltpu.SemaphoreType.DMA,
    ],
)


def _deg_body(dst_hbm, ones_hbm, zeros_hbm, out_hbm, dst_v, ones_v, acc_sh):
    c = lax.axis_index("c")
    s = lax.axis_index("s")
    wid = c * NS + s
    pltpu.sync_copy(zeros_hbm, acc_sh.at[pl.ds(s * RPT, RPT)])
    pltpu.sync_copy(ones_hbm, ones_v)
    pltpu.sync_copy(dst_hbm.at[pl.ds(wid * NCH, NCH)], dst_v)
    plsc.subcore_barrier()

    def body(j, carry):
        pltpu.sync_copy(ones_v, acc_sh.at[dst_v.at[j]], add=True)
        return carry

    lax.fori_loop(0, NCH, body, 0)
    plsc.subcore_barrier()
    pltpu.sync_copy(
        acc_sh.at[pl.ds(s * RPT, RPT)], out_hbm.at[c, pl.ds(s * RPT, RPT)]
    )


_deg = pl.kernel(
    _deg_body,
    out_type=jax.ShapeDtypeStruct((NC, NP, D), jnp.float32),
    mesh=_MESH,
    scratch_types=[
        pltpu.VMEM((NCH, CB), jnp.int32),
        pltpu.VMEM((CB, D), jnp.float32),
        pltpu.VMEM_SHARED((NP, D), jnp.float32),
    ],
)


BM = 1024


def _dinv_deginv(dp):
    deg = 1.0 + dp[0, :, 0] + dp[1, :, 0]
    return lax.rsqrt(deg), 1.0 / deg


def _dense0_body(x_ref, w_ref, dp_ref, t_ref, g_ref):
    dinv, _ = _dinv_deginv(dp_ref[...])
    t = jnp.dot(x_ref[...], w_ref[...], preferred_element_type=jnp.float32)
    t_ref[...] = t
    g_ref[...] = t * dinv[:, None]


def _mid_body(acc_ref, t_ref, b_ref, dp_ref, w_ref, tn_ref, gn_ref):
    dinv, deginv = _dinv_deginv(dp_ref[...])
    agg = ((acc_ref[0] + acc_ref[1]) * dinv[:, None]
           + t_ref[...] * deginv[:, None] + b_ref[...])
    h = jax.nn.sigmoid(agg)
    t = jnp.dot(h, w_ref[...], preferred_element_type=jnp.float32)
    tn_ref[...] = t
    gn_ref[...] = t * dinv[:, None]


def _fin_body(acc_ref, t_ref, b_ref, dp_ref, o_ref):
    dinv, deginv = _dinv_deginv(dp_ref[...])
    o_ref[...] = ((acc_ref[0] + acc_ref[1]) * dinv[:, None]
                  + t_ref[...] * deginv[:, None] + b_ref[...])


_spec_rows = pl.BlockSpec((BM, D), lambda i: (i, 0))
_spec_w = pl.BlockSpec((D, D), lambda i: (0, 0))
_spec_b = pl.BlockSpec((1, D), lambda i: (0, 0))
_spec_dp = pl.BlockSpec((NC, BM, DW), lambda i: (0, i, 0))
_spec_acc = pl.BlockSpec((NC, BM, D), lambda i: (0, i, 0))
_GRID = (NP // BM,)
_row_ty = jax.ShapeDtypeStruct((NP, D), jnp.float32)

_dense0 = pl.pallas_call(
    _dense0_body,
    grid=_GRID,
    in_specs=[_spec_rows, _spec_w, _spec_dp],
    out_specs=[_spec_rows, _spec_rows],
    out_shape=[_row_ty, _row_ty],
)

_mid = pl.pallas_call(
    _mid_body,
    grid=_GRID,
    in_specs=[_spec_acc, _spec_rows, _spec_b, _spec_dp, _spec_w],
    out_specs=[_spec_rows, _spec_rows],
    out_shape=[_row_ty, _row_ty],
)

_fin = pl.pallas_call(
    _fin_body,
    grid=_GRID,
    in_specs=[_spec_acc, _spec_rows, _spec_b, _spec_dp],
    out_specs=_spec_rows,
    out_shape=_row_ty,
)


@jax.jit
def kernel(x, edge_attr, edge_index, W0, b0, W1, b1, W2, b2):
    del edge_attr
    src = edge_index[0]
    dst = edge_index[1]
    pad = EP - E
    pad_dst = N + jnp.arange(pad, dtype=jnp.int32) % (NP - N)
    src_p = jnp.concatenate(
        [src, jnp.zeros((pad,), jnp.int32)]).reshape(TCH, CB)
    dst_p = jnp.concatenate([dst, pad_dst]).reshape(TCH, CB)
    x_p = jnp.pad(x, ((0, NP - N), (0, 0)))
    zeros_rows = jnp.zeros((RPT, D), jnp.float32)
    ones_rows = jnp.ones((CB, D), jnp.float32)

    dp = _deg(dst_p, ones_rows, zeros_rows)
    t0, g0 = _dense0(x_p, W0, dp)
    a0 = _prop(g0, src_p, dst_p, zeros_rows)
    t1, g1 = _mid(a0, t0, b0.reshape(1, D), dp, W1)
    a1 = _prop(g1, src_p, dst_p, zeros_rows)
    t2, g2 = _mid(a1, t1, b1.reshape(1, D), dp, W2)
    a2 = _prop(g2, src_p, dst_p, zeros_rows)
    out = _fin(a2, t2, b2.reshape(1, D), dp)
    return out[:N]


# speedup vs baseline: 2.0845x; 1.6890x over previous
import functools

import jax
import jax.numpy as jnp
from jax import lax
from jax.experimental import pallas as pl
from jax.experimental.pallas import tpu as pltpu
from jax.experimental.pallas import tpu_sc as plsc

N = 10000
NP = 10240
D = 128
E = 320000
NC, NS = 2, 16
NW = NC * NS
CB = 128
NCH = 80
TCH = NW * NCH
EP = TCH * CB
ACH = 112
BCH = NCH * 2 - ACH
NCH_MAX = max(ACH, BCH)
RPT = NP // NS
DW = 128

_MESH = plsc.VectorSubcoreMesh(
    core_axis_name="c", subcore_axis_name="s", num_cores=NC, num_subcores=NS
)


def _prop_body(g_hbm, src_hbm, dst_hbm, zeros_hbm, out_hbm,
               src_v, dst_v, rows_v, acc_sh, gsem):
    c = lax.axis_index("c")
    s = lax.axis_index("s")
    nch = jnp.where(c == 0, ACH, BCH)
    off = jnp.where(c == 0, s * ACH, 16 * ACH + s * BCH)
    pltpu.sync_copy(zeros_hbm, acc_sh.at[pl.ds(s * RPT, RPT)])
    pltpu.sync_copy(src_hbm.at[pl.ds(off, NCH_MAX)], src_v)
    pltpu.sync_copy(dst_hbm.at[pl.ds(off, NCH_MAX)], dst_v)
    plsc.subcore_barrier()

    def body(j, carry):
        pltpu.async_copy(g_hbm.at[src_v.at[j]], rows_v, gsem).wait()
        pltpu.sync_copy(rows_v, acc_sh.at[dst_v.at[j]], add=True)
        return carry

    lax.fori_loop(0, nch, body, 0)
    plsc.subcore_barrier()
    pltpu.sync_copy(
        acc_sh.at[pl.ds(s * RPT, RPT)], out_hbm.at[c, pl.ds(s * RPT, RPT)]
    )


_prop = pl.kernel(
    _prop_body,
    out_type=jax.ShapeDtypeStruct((NC, NP, D), jnp.float32),
    mesh=_MESH,
    scratch_types=[
        pltpu.VMEM((NCH_MAX, CB), jnp.int32),
        pltpu.VMEM((NCH_MAX, CB), jnp.int32),
        pltpu.VMEM((CB, D), jnp.float32),
        pltpu.VMEM_SHARED((NP, D), jnp.float32),
        pltpu.SemaphoreType.DMA,
    ],
)


def _deg_body(dst_hbm, ones_hbm, zeros_hbm, out_hbm, dst_v, ones_v, acc_sh):
    c = lax.axis_index("c")
    s = lax.axis_index("s")
    wid = c * NS + s
    pltpu.sync_copy(zeros_hbm, acc_sh.at[pl.ds(s * RPT, RPT)])
    pltpu.sync_copy(ones_hbm, ones_v)
    pltpu.sync_copy(dst_hbm.at[pl.ds(wid * NCH, NCH)], dst_v)
    plsc.subcore_barrier()

    def body(j, carry):
        pltpu.sync_copy(ones_v, acc_sh.at[dst_v.at[j]], add=True)
        return carry

    lax.fori_loop(0, NCH, body, 0)
    plsc.subcore_barrier()
    pltpu.sync_copy(
        acc_sh.at[pl.ds(s * RPT, RPT)], out_hbm.at[c, pl.ds(s * RPT, RPT)]
    )


_deg = pl.kernel(
    _deg_body,
    out_type=jax.ShapeDtypeStruct((NC, NP, D), jnp.float32),
    mesh=_MESH,
    scratch_types=[
        pltpu.VMEM((NCH, CB), jnp.int32),
        pltpu.VMEM((CB, D), jnp.float32),
        pltpu.VMEM_SHARED((NP, D), jnp.float32),
    ],
)


BM = 1024


def _dinv_deginv(dp):
    deg = 1.0 + dp[0, :, 0] + dp[1, :, 0]
    return lax.rsqrt(deg), 1.0 / deg


def _dense0_body(x_ref, w_ref, dp_ref, t_ref, g_ref):
    dinv, _ = _dinv_deginv(dp_ref[...])
    t = jnp.dot(x_ref[...], w_ref[...], preferred_element_type=jnp.float32)
    t_ref[...] = t
    g_ref[...] = t * dinv[:, None]


def _mid_body(acc_ref, t_ref, b_ref, dp_ref, w_ref, tn_ref, gn_ref):
    dinv, deginv = _dinv_deginv(dp_ref[...])
    agg = ((acc_ref[0] + acc_ref[1]) * dinv[:, None]
           + t_ref[...] * deginv[:, None] + b_ref[...])
    h = jax.nn.sigmoid(agg)
    t = jnp.dot(h, w_ref[...], preferred_element_type=jnp.float32)
    tn_ref[...] = t
    gn_ref[...] = t * dinv[:, None]


def _fin_body(acc_ref, t_ref, b_ref, dp_ref, o_ref):
    dinv, deginv = _dinv_deginv(dp_ref[...])
    o_ref[...] = ((acc_ref[0] + acc_ref[1]) * dinv[:, None]
                  + t_ref[...] * deginv[:, None] + b_ref[...])


_spec_rows = pl.BlockSpec((BM, D), lambda i: (i, 0))
_spec_w = pl.BlockSpec((D, D), lambda i: (0, 0))
_spec_b = pl.BlockSpec((1, D), lambda i: (0, 0))
_spec_dp = pl.BlockSpec((NC, BM, DW), lambda i: (0, i, 0))
_spec_acc = pl.BlockSpec((NC, BM, D), lambda i: (0, i, 0))
_GRID = (NP // BM,)
_row_ty = jax.ShapeDtypeStruct((NP, D), jnp.float32)

_dense0 = pl.pallas_call(
    _dense0_body,
    grid=_GRID,
    in_specs=[_spec_rows, _spec_w, _spec_dp],
    out_specs=[_spec_rows, _spec_rows],
    out_shape=[_row_ty, _row_ty],
)

_mid = pl.pallas_call(
    _mid_body,
    grid=_GRID,
    in_specs=[_spec_acc, _spec_rows, _spec_b, _spec_dp, _spec_w],
    out_specs=[_spec_rows, _spec_rows],
    out_shape=[_row_ty, _row_ty],
)

_fin = pl.pallas_call(
    _fin_body,
    grid=_GRID,
    in_specs=[_spec_acc, _spec_rows, _spec_b, _spec_dp],
    out_specs=_spec_rows,
    out_shape=_row_ty,
)


@jax.jit
def kernel(x, edge_attr, edge_index, W0, b0, W1, b1, W2, b2):
    del edge_attr
    src = edge_index[0]
    dst = edge_index[1]
    pad = EP - E
    pad_dst = N + jnp.arange(pad, dtype=jnp.int32) % (NP - N)
    pad_src = (jnp.arange(pad, dtype=jnp.int32) * 131) % N
    src_p = jnp.concatenate([src, pad_src]).reshape(TCH, CB)
    dst_p = jnp.concatenate([dst, pad_dst]).reshape(TCH, CB)
    x_p = jnp.pad(x, ((0, NP - N), (0, 0)))
    zeros_rows = jnp.zeros((RPT, D), jnp.float32)
    ones_rows = jnp.ones((CB, D), jnp.float32)

    dp = _deg(dst_p, ones_rows, zeros_rows)
    t0, g0 = _dense0(x_p, W0, dp)
    a0 = _prop(g0, src_p, dst_p, zeros_rows)
    t1, g1 = _mid(a0, t0, b0.reshape(1, D), dp, W1)
    a1 = _prop(g1, src_p, dst_p, zeros_rows)
    t2, g2 = _mid(a1, t1, b1.reshape(1, D), dp, W2)
    a2 = _prop(g2, src_p, dst_p, zeros_rows)
    out = _fin(a2, t2, b2.reshape(1, D), dp)
    return out[:N]


# speedup vs baseline: 2.4769x; 1.1883x over previous
import functools

import jax
import jax.numpy as jnp
from jax import lax
from jax.experimental import pallas as pl
from jax.experimental.pallas import tpu as pltpu
from jax.experimental.pallas import tpu_sc as plsc

N = 10000
NP = 10240
D = 128
E = 320000
NC, NS = 2, 16
NW = NC * NS
CB = 128
NCH = 80
TCH = NW * NCH
EP = TCH * CB
ACH = 88
BCH = NCH * 2 - ACH
NCH_MAX = max(ACH, BCH)
RPT = NP // NS
DW = 128

_MESH = plsc.VectorSubcoreMesh(
    core_axis_name="c", subcore_axis_name="s", num_cores=NC, num_subcores=NS
)


def _prop_body(g_hbm, src_hbm, dst_hbm, zeros_hbm, out_hbm,
               src_v, dst_v, rows_v, acc_sh, gsem):
    c = lax.axis_index("c")
    s = lax.axis_index("s")
    nch = jnp.where(c == 0, ACH, BCH)
    off = jnp.where(c == 0, s * ACH, 16 * ACH + s * BCH)
    pltpu.sync_copy(zeros_hbm, acc_sh.at[pl.ds(s * RPT, RPT)])
    pltpu.sync_copy(src_hbm.at[pl.ds(off, NCH_MAX)], src_v)
    pltpu.sync_copy(dst_hbm.at[pl.ds(off, NCH_MAX)], dst_v)
    plsc.subcore_barrier()

    def body(j, carry):
        pltpu.async_copy(g_hbm.at[src_v.at[j]], rows_v, gsem).wait()
        pltpu.sync_copy(rows_v, acc_sh.at[dst_v.at[j]], add=True)
        return carry

    lax.fori_loop(0, nch, body, 0)
    plsc.subcore_barrier()
    pltpu.sync_copy(
        acc_sh.at[pl.ds(s * RPT, RPT)], out_hbm.at[c, pl.ds(s * RPT, RPT)]
    )


_prop = pl.kernel(
    _prop_body,
    out_type=jax.ShapeDtypeStruct((NC, NP, D), jnp.float32),
    mesh=_MESH,
    scratch_types=[
        pltpu.VMEM((NCH_MAX, CB), jnp.int32),
        pltpu.VMEM((NCH_MAX, CB), jnp.int32),
        pltpu.VMEM((CB, D), jnp.float32),
        pltpu.VMEM_SHARED((NP, D), jnp.float32),
        pltpu.SemaphoreType.DMA,
    ],
)


def _deg_body(dst_hbm, ones_hbm, zeros_hbm, out_hbm, dst_v, ones_v, acc_sh):
    c = lax.axis_index("c")
    s = lax.axis_index("s")
    wid = c * NS + s
    pltpu.sync_copy(zeros_hbm, acc_sh.at[pl.ds(s * RPT, RPT)])
    pltpu.sync_copy(ones_hbm, ones_v)
    pltpu.sync_copy(dst_hbm.at[pl.ds(wid * NCH, NCH)], dst_v)
    plsc.subcore_barrier()

    def body(j, carry):
        pltpu.sync_copy(ones_v, acc_sh.at[dst_v.at[j]], add=True)
        return carry

    lax.fori_loop(0, NCH, body, 0)
    plsc.subcore_barrier()
    pltpu.sync_copy(
        acc_sh.at[pl.ds(s * RPT, RPT)], out_hbm.at[c, pl.ds(s * RPT, RPT)]
    )


_deg = pl.kernel(
    _deg_body,
    out_type=jax.ShapeDtypeStruct((NC, NP, D), jnp.float32),
    mesh=_MESH,
    scratch_types=[
        pltpu.VMEM((NCH, CB), jnp.int32),
        pltpu.VMEM((CB, D), jnp.float32),
        pltpu.VMEM_SHARED((NP, D), jnp.float32),
    ],
)


BM = 1024


def _dinv_deginv(dp):
    deg = 1.0 + dp[0, :, 0] + dp[1, :, 0]
    return lax.rsqrt(deg), 1.0 / deg


def _dense0_body(x_ref, w_ref, dp_ref, t_ref, g_ref):
    dinv, _ = _dinv_deginv(dp_ref[...])
    t = jnp.dot(x_ref[...], w_ref[...], preferred_element_type=jnp.float32)
    t_ref[...] = t
    g_ref[...] = t * dinv[:, None]


def _mid_body(acc_ref, t_ref, b_ref, dp_ref, w_ref, tn_ref, gn_ref):
    dinv, deginv = _dinv_deginv(dp_ref[...])
    agg = ((acc_ref[0] + acc_ref[1]) * dinv[:, None]
           + t_ref[...] * deginv[:, None] + b_ref[...])
    h = jax.nn.sigmoid(agg)
    t = jnp.dot(h, w_ref[...], preferred_element_type=jnp.float32)
    tn_ref[...] = t
    gn_ref[...] = t * dinv[:, None]


def _fin_body(acc_ref, t_ref, b_ref, dp_ref, o_ref):
    dinv, deginv = _dinv_deginv(dp_ref[...])
    o_ref[...] = ((acc_ref[0] + acc_ref[1]) * dinv[:, None]
                  + t_ref[...] * deginv[:, None] + b_ref[...])


_spec_rows = pl.BlockSpec((BM, D), lambda i: (i, 0))
_spec_w = pl.BlockSpec((D, D), lambda i: (0, 0))
_spec_b = pl.BlockSpec((1, D), lambda i: (0, 0))
_spec_dp = pl.BlockSpec((NC, BM, DW), lambda i: (0, i, 0))
_spec_acc = pl.BlockSpec((NC, BM, D), lambda i: (0, i, 0))
_GRID = (NP // BM,)
_row_ty = jax.ShapeDtypeStruct((NP, D), jnp.float32)

_dense0 = pl.pallas_call(
    _dense0_body,
    grid=_GRID,
    in_specs=[_spec_rows, _spec_w, _spec_dp],
    out_specs=[_spec_rows, _spec_rows],
    out_shape=[_row_ty, _row_ty],
)

_mid = pl.pallas_call(
    _mid_body,
    grid=_GRID,
    in_specs=[_spec_acc, _spec_rows, _spec_b, _spec_dp, _spec_w],
    out_specs=[_spec_rows, _spec_rows],
    out_shape=[_row_ty, _row_ty],
)

_fin = pl.pallas_call(
    _fin_body,
    grid=_GRID,
    in_specs=[_spec_acc, _spec_rows, _spec_b, _spec_dp],
    out_specs=_spec_rows,
    out_shape=_row_ty,
)


@jax.jit
def kernel(x, edge_attr, edge_index, W0, b0, W1, b1, W2, b2):
    del edge_attr
    src = edge_index[0]
    dst = edge_index[1]
    pad = EP - E
    pad_dst = N + jnp.arange(pad, dtype=jnp.int32) % (NP - N)
    pad_src = (jnp.arange(pad, dtype=jnp.int32) * 131) % N
    src_p = jnp.concatenate([src, pad_src]).reshape(TCH, CB)
    dst_p = jnp.concatenate([dst, pad_dst]).reshape(TCH, CB)
    x_p = jnp.pad(x, ((0, NP - N), (0, 0)))
    zeros_rows = jnp.zeros((RPT, D), jnp.float32)
    ones_rows = jnp.ones((CB, D), jnp.float32)

    dp = _deg(dst_p, ones_rows, zeros_rows)
    t0, g0 = _dense0(x_p, W0, dp)
    a0 = _prop(g0, src_p, dst_p, zeros_rows)
    t1, g1 = _mid(a0, t0, b0.reshape(1, D), dp, W1)
    a1 = _prop(g1, src_p, dst_p, zeros_rows)
    t2, g2 = _mid(a1, t1, b1.reshape(1, D), dp, W2)
    a2 = _prop(g2, src_p, dst_p, zeros_rows)
    out = _fin(a2, t2, b2.reshape(1, D), dp)
    return out[:N]


# speedup vs baseline: 2.6362x; 1.0643x over previous
import functools

import jax
import jax.numpy as jnp
from jax import lax
from jax.experimental import pallas as pl
from jax.experimental.pallas import tpu as pltpu
from jax.experimental.pallas import tpu_sc as plsc

N = 10000
NP = 10240
D = 128
E = 320000
NC, NS = 2, 16
NW = NC * NS
CB = 128
NCH = 80
TCH = NW * NCH
EP = TCH * CB
ACH = 80
BCH = NCH * 2 - ACH
NCH_MAX = max(ACH, BCH)
RPT = NP // NS
DW = 128

_MESH = plsc.VectorSubcoreMesh(
    core_axis_name="c", subcore_axis_name="s", num_cores=NC, num_subcores=NS
)


def _prop_body(g_hbm, src_hbm, dst_hbm, zeros_hbm, out_hbm,
               src_v, dst_v, rows_v, acc_sh, gsem):
    c = lax.axis_index("c")
    s = lax.axis_index("s")
    nch = jnp.where(c == 0, ACH, BCH)
    off = jnp.where(c == 0, s * ACH, 16 * ACH + s * BCH)
    pltpu.sync_copy(zeros_hbm, acc_sh.at[pl.ds(s * RPT, RPT)])
    pltpu.sync_copy(src_hbm.at[pl.ds(off, NCH_MAX)], src_v)
    pltpu.sync_copy(dst_hbm.at[pl.ds(off, NCH_MAX)], dst_v)
    plsc.subcore_barrier()

    def body(j, carry):
        pltpu.async_copy(g_hbm.at[src_v.at[j]], rows_v, gsem).wait()
        pltpu.sync_copy(rows_v, acc_sh.at[dst_v.at[j]], add=True)
        return carry

    lax.fori_loop(0, nch, body, 0)
    plsc.subcore_barrier()
    pltpu.sync_copy(
        acc_sh.at[pl.ds(s * RPT, RPT)], out_hbm.at[c, pl.ds(s * RPT, RPT)]
    )


_prop = pl.kernel(
    _prop_body,
    out_type=jax.ShapeDtypeStruct((NC, NP, D), jnp.float32),
    mesh=_MESH,
    scratch_types=[
        pltpu.VMEM((NCH_MAX, CB), jnp.int32),
        pltpu.VMEM((NCH_MAX, CB), jnp.int32),
        pltpu.VMEM((CB, D), jnp.float32),
        pltpu.VMEM_SHARED((NP, D), jnp.float32),
        pltpu.SemaphoreType.DMA,
    ],
)


def _deg_body(dst_hbm, ones_hbm, zeros_hbm, out_hbm, dst_v, ones_v, acc_sh):
    c = lax.axis_index("c")
    s = lax.axis_index("s")
    wid = c * NS + s
    pltpu.sync_copy(zeros_hbm, acc_sh.at[pl.ds(s * RPT, RPT)])
    pltpu.sync_copy(ones_hbm, ones_v)
    pltpu.sync_copy(dst_hbm.at[pl.ds(wid * NCH, NCH)], dst_v)
    plsc.subcore_barrier()

    def body(j, carry):
        pltpu.sync_copy(ones_v, acc_sh.at[dst_v.at[j]], add=True)
        return carry

    lax.fori_loop(0, NCH, body, 0)
    plsc.subcore_barrier()
    pltpu.sync_copy(
        acc_sh.at[pl.ds(s * RPT, RPT)], out_hbm.at[c, pl.ds(s * RPT, RPT)]
    )


_deg = pl.kernel(
    _deg_body,
    out_type=jax.ShapeDtypeStruct((NC, NP, D), jnp.float32),
    mesh=_MESH,
    scratch_types=[
        pltpu.VMEM((NCH, CB), jnp.int32),
        pltpu.VMEM((CB, D), jnp.float32),
        pltpu.VMEM_SHARED((NP, D), jnp.float32),
    ],
)


BM = 1024


def _dinv_deginv(dp):
    deg = 1.0 + dp[0, :, 0] + dp[1, :, 0]
    return lax.rsqrt(deg), 1.0 / deg


def _dense0_body(x_ref, w_ref, dp_ref, t_ref, g_ref):
    dinv, _ = _dinv_deginv(dp_ref[...])
    t = jnp.dot(x_ref[...], w_ref[...], preferred_element_type=jnp.float32)
    t_ref[...] = t
    g_ref[...] = t * dinv[:, None]


def _mid_body(acc_ref, t_ref, b_ref, dp_ref, w_ref, tn_ref, gn_ref):
    dinv, deginv = _dinv_deginv(dp_ref[...])
    agg = ((acc_ref[0] + acc_ref[1]) * dinv[:, None]
           + t_ref[...] * deginv[:, None] + b_ref[...])
    h = jax.nn.sigmoid(agg)
    t = jnp.dot(h, w_ref[...], preferred_element_type=jnp.float32)
    tn_ref[...] = t
    gn_ref[...] = t * dinv[:, None]


def _fin_body(acc_ref, t_ref, b_ref, dp_ref, o_ref):
    dinv, deginv = _dinv_deginv(dp_ref[...])
    o_ref[...] = ((acc_ref[0] + acc_ref[1]) * dinv[:, None]
                  + t_ref[...] * deginv[:, None] + b_ref[...])


_spec_rows = pl.BlockSpec((BM, D), lambda i: (i, 0))
_spec_w = pl.BlockSpec((D, D), lambda i: (0, 0))
_spec_b = pl.BlockSpec((1, D), lambda i: (0, 0))
_spec_dp = pl.BlockSpec((NC, BM, DW), lambda i: (0, i, 0))
_spec_acc = pl.BlockSpec((NC, BM, D), lambda i: (0, i, 0))
_GRID = (NP // BM,)
_row_ty = jax.ShapeDtypeStruct((NP, D), jnp.float32)

_dense0 = pl.pallas_call(
    _dense0_body,
    grid=_GRID,
    in_specs=[_spec_rows, _spec_w, _spec_dp],
    out_specs=[_spec_rows, _spec_rows],
    out_shape=[_row_ty, _row_ty],
)

_mid = pl.pallas_call(
    _mid_body,
    grid=_GRID,
    in_specs=[_spec_acc, _spec_rows, _spec_b, _spec_dp, _spec_w],
    out_specs=[_spec_rows, _spec_rows],
    out_shape=[_row_ty, _row_ty],
)

_fin = pl.pallas_call(
    _fin_body,
    grid=_GRID,
    in_specs=[_spec_acc, _spec_rows, _spec_b, _spec_dp],
    out_specs=_spec_rows,
    out_shape=_row_ty,
)


@jax.jit
def kernel(x, edge_attr, edge_index, W0, b0, W1, b1, W2, b2):
    del edge_attr
    src = edge_index[0]
    dst = edge_index[1]
    pad = EP - E
    pad_dst = N + jnp.arange(pad, dtype=jnp.int32) % (NP - N)
    pad_src = (jnp.arange(pad, dtype=jnp.int32) * 131) % N
    src_p = jnp.concatenate([src, pad_src]).reshape(TCH, CB)
    dst_p = jnp.concatenate([dst, pad_dst]).reshape(TCH, CB)
    x_p = jnp.pad(x, ((0, NP - N), (0, 0)))
    zeros_rows = jnp.zeros((RPT, D), jnp.float32)
    ones_rows = jnp.ones((CB, D), jnp.float32)

    dp = _deg(dst_p, ones_rows, zeros_rows)
    t0, g0 = _dense0(x_p, W0, dp)
    a0 = _prop(g0, src_p, dst_p, zeros_rows)
    t1, g1 = _mid(a0, t0, b0.reshape(1, D), dp, W1)
    a1 = _prop(g1, src_p, dst_p, zeros_rows)
    t2, g2 = _mid(a1, t1, b1.reshape(1, D), dp, W2)
    a2 = _prop(g2, src_p, dst_p, zeros_rows)
    out = _fin(a2, t2, b2.reshape(1, D), dp)
    return out[:N]


# speedup vs baseline: 3.3104x; 1.2557x over previous
import functools

import jax
import jax.numpy as jnp
from jax import lax
from jax.experimental import pallas as pl
from jax.experimental.pallas import tpu as pltpu
from jax.experimental.pallas import tpu_sc as plsc

N = 10000
NP = 10240
D = 128
E = 320000
NC, NS = 2, 16
NW = NC * NS
CB = 128
NCH = 80
TCH = NW * NCH
EP = TCH * CB
ACH = 80
BCH = NCH * 2 - ACH
NCH_MAX = max(ACH, BCH)
RPT = NP // NS
DW = 128

_MESH = plsc.VectorSubcoreMesh(
    core_axis_name="c", subcore_axis_name="s", num_cores=NC, num_subcores=NS
)


def _prop_body(g_hbm, src_hbm, dst_hbm, zeros_hbm, out_hbm,
               srcb, dst_v, rows_v, acc_sh, gsem0, gsem1, isem0, isem1):
    gsem = (gsem0, gsem1)
    isem = (isem0, isem1)
    c = lax.axis_index("c")
    s = lax.axis_index("s")
    off = (c * NS + s) * NCH

    def start_src(j, b):
        pltpu.async_copy(src_hbm.at[off + j], srcb.at[b], isem[b])

    def wait_src(j, b):
        pltpu.make_async_copy(src_hbm.at[off + j], srcb.at[b],
                              isem[b]).wait()

    def start_gather(b):
        pltpu.async_copy(g_hbm.at[srcb.at[b]], rows_v.at[b], gsem[b])

    def wait_gather(b):
        pltpu.make_async_copy(g_hbm.at[srcb.at[b]], rows_v.at[b],
                              gsem[b]).wait()

    start_src(0, 0)
    pltpu.sync_copy(zeros_hbm, acc_sh.at[pl.ds(s * RPT, RPT)])
    pltpu.sync_copy(dst_hbm.at[pl.ds(off, NCH)], dst_v)
    plsc.subcore_barrier()
    wait_src(0, 0)
    start_gather(0)
    start_src(1, 1)

    def step(j, b, more_gather, more_src):
        wait_gather(b)
        if more_gather:
            wait_src(j + 1, 1 - b)
            start_gather(1 - b)
        pltpu.sync_copy(rows_v.at[b], acc_sh.at[dst_v.at[j]], add=True)
        if more_src:
            start_src(j + 2, b)

    def middle(k, carry):
        j0 = k * 2
        step(j0, 0, more_gather=True, more_src=True)
        step(j0 + 1, 1, more_gather=True, more_src=True)
        return carry

    lax.fori_loop(0, NCH // 2 - 1, middle, 0)
    step(NCH - 2, 0, more_gather=True, more_src=False)
    step(NCH - 1, 1, more_gather=False, more_src=False)

    plsc.subcore_barrier()
    pltpu.sync_copy(
        acc_sh.at[pl.ds(s * RPT, RPT)], out_hbm.at[c, pl.ds(s * RPT, RPT)]
    )


_prop = pl.kernel(
    _prop_body,
    out_type=jax.ShapeDtypeStruct((NC, NP, D), jnp.float32),
    mesh=_MESH,
    scratch_types=[
        pltpu.VMEM((2, CB), jnp.int32),
        pltpu.VMEM((NCH, CB), jnp.int32),
        pltpu.VMEM((2, CB, D), jnp.float32),
        pltpu.VMEM_SHARED((NP, D), jnp.float32),
        pltpu.SemaphoreType.DMA,
        pltpu.SemaphoreType.DMA,
        pltpu.SemaphoreType.DMA,
        pltpu.SemaphoreType.DMA,
    ],
)


def _deg_body(dst_hbm, ones_hbm, zeros_hbm, out_hbm, dst_v, ones_v, acc_sh):
    c = lax.axis_index("c")
    s = lax.axis_index("s")
    wid = c * NS + s
    pltpu.sync_copy(zeros_hbm, acc_sh.at[pl.ds(s * RPT, RPT)])
    pltpu.sync_copy(ones_hbm, ones_v)
    pltpu.sync_copy(dst_hbm.at[pl.ds(wid * NCH, NCH)], dst_v)
    plsc.subcore_barrier()

    def body(j, carry):
        pltpu.sync_copy(ones_v, acc_sh.at[dst_v.at[j]], add=True)
        return carry

    lax.fori_loop(0, NCH, body, 0)
    plsc.subcore_barrier()
    pltpu.sync_copy(
        acc_sh.at[pl.ds(s * RPT, RPT)], out_hbm.at[c, pl.ds(s * RPT, RPT)]
    )


_deg = pl.kernel(
    _deg_body,
    out_type=jax.ShapeDtypeStruct((NC, NP, D), jnp.float32),
    mesh=_MESH,
    scratch_types=[
        pltpu.VMEM((NCH, CB), jnp.int32),
        pltpu.VMEM((CB, D), jnp.float32),
        pltpu.VMEM_SHARED((NP, D), jnp.float32),
    ],
)


BM = 1024


def _dinv_deginv(dp):
    deg = 1.0 + dp[0, :, 0] + dp[1, :, 0]
    return lax.rsqrt(deg), 1.0 / deg


def _dense0_body(x_ref, w_ref, dp_ref, t_ref, g_ref):
    dinv, _ = _dinv_deginv(dp_ref[...])
    t = jnp.dot(x_ref[...], w_ref[...], preferred_element_type=jnp.float32)
    t_ref[...] = t
    g_ref[...] = t * dinv[:, None]


def _mid_body(acc_ref, t_ref, b_ref, dp_ref, w_ref, tn_ref, gn_ref):
    dinv, deginv = _dinv_deginv(dp_ref[...])
    agg = ((acc_ref[0] + acc_ref[1]) * dinv[:, None]
           + t_ref[...] * deginv[:, None] + b_ref[...])
    h = jax.nn.sigmoid(agg)
    t = jnp.dot(h, w_ref[...], preferred_element_type=jnp.float32)
    tn_ref[...] = t
    gn_ref[...] = t * dinv[:, None]


def _fin_body(acc_ref, t_ref, b_ref, dp_ref, o_ref):
    dinv, deginv = _dinv_deginv(dp_ref[...])
    o_ref[...] = ((acc_ref[0] + acc_ref[1]) * dinv[:, None]
                  + t_ref[...] * deginv[:, None] + b_ref[...])


_spec_rows = pl.BlockSpec((BM, D), lambda i: (i, 0))
_spec_w = pl.BlockSpec((D, D), lambda i: (0, 0))
_spec_b = pl.BlockSpec((1, D), lambda i: (0, 0))
_spec_dp = pl.BlockSpec((NC, BM, DW), lambda i: (0, i, 0))
_spec_acc = pl.BlockSpec((NC, BM, D), lambda i: (0, i, 0))
_GRID = (NP // BM,)
_row_ty = jax.ShapeDtypeStruct((NP, D), jnp.float32)

_dense0 = pl.pallas_call(
    _dense0_body,
    grid=_GRID,
    in_specs=[_spec_rows, _spec_w, _spec_dp],
    out_specs=[_spec_rows, _spec_rows],
    out_shape=[_row_ty, _row_ty],
)

_mid = pl.pallas_call(
    _mid_body,
    grid=_GRID,
    in_specs=[_spec_acc, _spec_rows, _spec_b, _spec_dp, _spec_w],
    out_specs=[_spec_rows, _spec_rows],
    out_shape=[_row_ty, _row_ty],
)

_fin = pl.pallas_call(
    _fin_body,
    grid=_GRID,
    in_specs=[_spec_acc, _spec_rows, _spec_b, _spec_dp],
    out_specs=_spec_rows,
    out_shape=_row_ty,
)


@jax.jit
def kernel(x, edge_attr, edge_index, W0, b0, W1, b1, W2, b2):
    del edge_attr
    src = edge_index[0]
    dst = edge_index[1]
    pad = EP - E
    pad_dst = N + jnp.arange(pad, dtype=jnp.int32) % (NP - N)
    pad_src = (jnp.arange(pad, dtype=jnp.int32) * 131) % N
    src_p = jnp.concatenate([src, pad_src]).reshape(TCH, CB)
    dst_p = jnp.concatenate([dst, pad_dst]).reshape(TCH, CB)
    x_p = jnp.pad(x, ((0, NP - N), (0, 0)))
    zeros_rows = jnp.zeros((RPT, D), jnp.float32)
    ones_rows = jnp.ones((CB, D), jnp.float32)

    dp = _deg(dst_p, ones_rows, zeros_rows)
    t0, g0 = _dense0(x_p, W0, dp)
    a0 = _prop(g0, src_p, dst_p, zeros_rows)
    t1, g1 = _mid(a0, t0, b0.reshape(1, D), dp, W1)
    a1 = _prop(g1, src_p, dst_p, zeros_rows)
    t2, g2 = _mid(a1, t1, b1.reshape(1, D), dp, W2)
    a2 = _prop(g2, src_p, dst_p, zeros_rows)
    out = _fin(a2, t2, b2.reshape(1, D), dp)
    return out[:N]
